# baseline (speedup 1.0000x reference)
import functools

import jax
import jax.numpy as jnp
from jax import lax
from jax.experimental import pallas as pl
from jax.experimental.pallas import tpu as pltpu
from jax.experimental.pallas import tpu_sc as plsc

N = 10000
D = 128
H = 128
C = 16
K = 4
RWS = 8

NW = 32
NPW = 320
NPAD = NW * NPW
GROUP = 16
GROUPS = NPW // GROUP
GROW = GROUP * RWS
EPN = NPAD * RWS

TC_BLK = 1024


def _lin_body(x_ref, w_ref, b_ref, walr_ref, balr_ref, xl_ref, alr_ref):
    t = jnp.dot(x_ref[...], w_ref[...], preferred_element_type=jnp.float32)
    t = t + b_ref[...]
    xl_ref[...] = t
    alr_ref[...] = (
        jnp.dot(t, walr_ref[...], preferred_element_type=jnp.float32)
        + balr_ref[...]
    )


def _tc_linear(xp, w, b, walr, balr):
    grid = (NPAD // TC_BLK,)
    return pl.pallas_call(
        _lin_body,
        grid=grid,
        in_specs=[
            pl.BlockSpec((TC_BLK, D), lambda i: (i, 0)),
            pl.BlockSpec((D, H), lambda i: (0, 0)),
            pl.BlockSpec((1, H), lambda i: (0, 0)),
            pl.BlockSpec((H, H), lambda i: (0, 0)),
            pl.BlockSpec((1, H), lambda i: (0, 0)),
        ],
        out_specs=[
            pl.BlockSpec((TC_BLK, H), lambda i: (i, 0)),
            pl.BlockSpec((TC_BLK, H), lambda i: (i, 0)),
        ],
        out_shape=[
            jax.ShapeDtypeStruct((NPAD, H), jnp.float32),
            jax.ShapeDtypeStruct((NPAD, H), jnp.float32),
        ],
    )(xp, w, b, walr, balr)


def _out_body(h_ref, w_ref, b_ref, o_ref):
    z = jnp.dot(h_ref[...], w_ref[...], preferred_element_type=jnp.float32)
    z = z + b_ref[...]
    col = lax.broadcasted_iota(jnp.int32, z.shape, 1)
    zm = jnp.where(col < C, z, -jnp.inf)
    mx = jnp.max(zm, axis=1, keepdims=True)
    lse = jnp.log(jnp.sum(jnp.exp(zm - mx), axis=1, keepdims=True)) + mx
    o_ref[...] = z - lse


def _tc_out(h, wout, bout):
    grid = (NPAD // TC_BLK,)
    return pl.pallas_call(
        _out_body,
        grid=grid,
        in_specs=[
            pl.BlockSpec((TC_BLK, H), lambda i: (i, 0)),
            pl.BlockSpec((H, H), lambda i: (0, 0)),
            pl.BlockSpec((1, H), lambda i: (0, 0)),
        ],
        out_specs=pl.BlockSpec((TC_BLK, H), lambda i: (i, 0)),
        out_shape=jax.ShapeDtypeStruct((NPAD, H), jnp.float32),
    )(h, wout, bout)


SUB = 5
SUBS = GROUPS // SUB
NTILE = NPAD // 16


def _sc_body(xl_hbm, al_hbm, ar_hbm, ends_hbm, att_hbm, out_hbm,
             ends_v, ar_v, al_v, rows_v, att_v, wbuf_v, out_v, xsh_v, sem0):
    c = lax.axis_index("c")
    s = lax.axis_index("s")
    wid = s * 2 + c

    pltpu.sync_copy(
        xl_hbm.at[pl.ds(s * NTILE, NTILE)],
        xsh_v.at[pl.ds(s * NTILE, NTILE)],
    )
    pltpu.sync_copy(ar_hbm, ar_v)
    pltpu.sync_copy(att_hbm, att_v)
    plsc.subcore_barrier()

    zeros16 = jnp.zeros((16,), jnp.int32)
    lanes = lax.iota(jnp.int32, 16)
    att0 = att_v[0, :]

    def sub_body(sub, _):
        gb = wid * GROUPS + sub * SUB
        nb = gb * GROUP
        pltpu.sync_copy(ends_hbm.at[pl.ds(gb, SUB)], ends_v)
        pltpu.sync_copy(al_hbm.at[pl.ds(nb, SUB * GROUP)], al_v)
        pltpu.sync_copy(xl_hbm.at[pl.ds(nb, SUB * GROUP)], out_v)

        def scale_body(r):
            for cc in range(H // 16):
                out_v[r, pl.ds(cc * 16, 16)] = (
                    out_v[r, pl.ds(cc * 16, 16)] * att0
                )

        plsc.parallel_loop(0, SUB * GROUP, 1, unroll=4)(scale_body)

        def step_body(t, _):
            g = t // K
            hop = t - g * K
            pltpu.async_copy(
                xsh_v.at[ends_v.at[g, hop]], rows_v, sem0
            ).wait()

            atth = att_v[hop + 1, :] * (1.0 / RWS)
            alv = al_v[pl.ds(g * GROUP, GROUP)]
            svs = []
            for w in range(RWS):
                pos = lanes * RWS + w
                idxw = plsc.load_gather(
                    ends_v, [zeros16 + g, zeros16 + hop, pos]
                )
                arw = plsc.load_gather(ar_v, [idxw])
                e = alv + arw
                svs.append(jnp.maximum(e, 0.2 * e))
            m = svs[0]
            for w in range(1, RWS):
                m = jnp.maximum(m, svs[w])
            ps = [jnp.exp(sv - m) for sv in svs]
            z = ps[0]
            for w in range(1, RWS):
                z = z + ps[w]
            coef = atth / z
            for w in range(RWS):
                wbuf_v[pl.ds(w * 16, 16)] = ps[w] * coef

            def n_body(n):
                row = g * GROUP + n
                wsp = [
                    plsc.load_gather(wbuf_v, [zeros16 + (w * 16) + n])
                    for w in range(RWS)
                ]
                for cc in range(H // 16):
                    t = [
                        wsp[w] * rows_v[n * RWS + w, pl.ds(cc * 16, 16)]
                        for w in range(RWS)
                    ]
                    t = [t[0] + t[1], t[2] + t[3], t[4] + t[5], t[6] + t[7]]
                    acc = (t[0] + t[1]) + (t[2] + t[3])
                    plsc.addupdate(out_v.at[row, pl.ds(cc * 16, 16)], acc)

            plsc.parallel_loop(0, GROUP, 1, unroll=4)(n_body)
            return 0

        lax.fori_loop(0, K * SUB, step_body, 0)
        pltpu.sync_copy(out_v, out_hbm.at[pl.ds(nb, SUB * GROUP)])
        return 0

    lax.fori_loop(0, SUBS, sub_body, 0)


@functools.lru_cache(maxsize=None)
def _sc_aggregate():
    return pl.kernel(
        _sc_body,
        out_type=jax.ShapeDtypeStruct((NPAD, H), jnp.float32),
        mesh=plsc.VectorSubcoreMesh(
            core_axis_name="c",
            subcore_axis_name="s",
            num_cores=2,
            num_subcores=16,
        ),
        scratch_types=[
            pltpu.VMEM((SUB, K, GROW), jnp.int32),
            pltpu.VMEM((NPAD,), jnp.float32),
            pltpu.VMEM((SUB * GROUP,), jnp.float32),
            pltpu.VMEM((GROW, H), jnp.float32),
            pltpu.VMEM((8, 16), jnp.float32),
            pltpu.VMEM((RWS * 16,), jnp.float32),
            pltpu.VMEM((SUB * GROUP, H), jnp.float32),
            pltpu.VMEM_SHARED((NPAD, H), jnp.float32),
            pltpu.SemaphoreType.DMA,
        ],
        compiler_params=pltpu.CompilerParams(needs_layout_passes=False),
    )


def _layer(xp, ends1d, attp, w, b, walr, balr):
    xl, alr = _tc_linear(xp, w, b, walr, balr)
    al = alr[:, 0]
    ar = alr[:, 1]
    return _sc_aggregate()(xl, al, ar, ends1d, attp)


def kernel(x, edge_index, walk_ends, att, W0, b0, Wal0, bal0, War0, bar0,
           W1, b1, Wal1, bal1, War1, bar1, Wout, bout):
    xp = jnp.zeros((NPAD, D), jnp.float32).at[:N].set(x)

    ends = walk_ends.astype(jnp.int32).reshape(2 * K, N * RWS)
    ends_pad = jnp.zeros((2 * K, EPN), jnp.int32).at[:, : N * RWS].set(ends)
    ends0 = jnp.transpose(
        ends_pad[:K].reshape(K, NPAD // GROUP, GROW), (1, 0, 2)
    )
    ends1 = jnp.transpose(
        ends_pad[K:].reshape(K, NPAD // GROUP, GROW), (1, 0, 2)
    )

    attp0 = jnp.zeros((8, 16), jnp.float32).at[: K + 1].set(
        jnp.tile(att[0][:, None], (1, 16))
    )
    attp1 = jnp.zeros((8, 16), jnp.float32).at[: K + 1].set(
        jnp.tile(att[1][:, None], (1, 16))
    )

    def packw(wal, bal, war, bar):
        walr = jnp.zeros((H, H), jnp.float32)
        walr = walr.at[:, 0].set(wal[:, 0]).at[:, 1].set(war[:, 0])
        balr = jnp.zeros((1, H), jnp.float32)
        balr = balr.at[0, 0].set(bal[0]).at[0, 1].set(bar[0])
        return walr, balr

    walr0, balr0 = packw(Wal0, bal0, War0, bar0)
    walr1, balr1 = packw(Wal1, bal1, War1, bar1)

    h = _layer(xp, ends0, attp0, W0, b0.reshape(1, H), walr0, balr0)
    h = _layer(h, ends1, attp1, W1, b1.reshape(1, H), walr1, balr1)

    woutp = jnp.zeros((H, H), jnp.float32).at[:, :C].set(Wout)
    boutp = jnp.zeros((1, H), jnp.float32).at[0, :C].set(bout)
    out = _tc_out(h, woutp, boutp)
    return out[:N, :C]


# speedup vs baseline: 1.0348x; 1.0348x over previous
import functools

import jax
import jax.numpy as jnp
from jax import lax
from jax.experimental import pallas as pl
from jax.experimental.pallas import tpu as pltpu
from jax.experimental.pallas import tpu_sc as plsc

N = 10000
D = 128
H = 128
C = 16
K = 4
RWS = 8

NW = 32
NPW = 320
NPAD = NW * NPW
GROUP = 16
GROUPS = NPW // GROUP
GROW = GROUP * RWS
EPN = NPAD * RWS

TC_BLK = 1024


def _lin_body(x_ref, w_ref, b_ref, walr_ref, balr_ref, xl_ref, alr_ref):
    t = jnp.dot(x_ref[...], w_ref[...], preferred_element_type=jnp.float32)
    t = t + b_ref[...]
    xl_ref[...] = t
    alr_ref[...] = (
        jnp.dot(t, walr_ref[...], preferred_element_type=jnp.float32)
        + balr_ref[...]
    )


def _tc_linear(xp, w, b, walr, balr):
    grid = (NPAD // TC_BLK,)
    return pl.pallas_call(
        _lin_body,
        grid=grid,
        in_specs=[
            pl.BlockSpec((TC_BLK, D), lambda i: (i, 0)),
            pl.BlockSpec((D, H), lambda i: (0, 0)),
            pl.BlockSpec((1, H), lambda i: (0, 0)),
            pl.BlockSpec((H, H), lambda i: (0, 0)),
            pl.BlockSpec((1, H), lambda i: (0, 0)),
        ],
        out_specs=[
            pl.BlockSpec((TC_BLK, H), lambda i: (i, 0)),
            pl.BlockSpec((TC_BLK, H), lambda i: (i, 0)),
        ],
        out_shape=[
            jax.ShapeDtypeStruct((NPAD, H), jnp.float32),
            jax.ShapeDtypeStruct((NPAD, H), jnp.float32),
        ],
    )(xp, w, b, walr, balr)


def _out_body(h_ref, w_ref, b_ref, o_ref):
    z = jnp.dot(h_ref[...], w_ref[...], preferred_element_type=jnp.float32)
    z = z + b_ref[...]
    col = lax.broadcasted_iota(jnp.int32, z.shape, 1)
    zm = jnp.where(col < C, z, -jnp.inf)
    mx = jnp.max(zm, axis=1, keepdims=True)
    lse = jnp.log(jnp.sum(jnp.exp(zm - mx), axis=1, keepdims=True)) + mx
    o_ref[...] = z - lse


def _tc_out(h, wout, bout):
    grid = (NPAD // TC_BLK,)
    return pl.pallas_call(
        _out_body,
        grid=grid,
        in_specs=[
            pl.BlockSpec((TC_BLK, H), lambda i: (i, 0)),
            pl.BlockSpec((H, H), lambda i: (0, 0)),
            pl.BlockSpec((1, H), lambda i: (0, 0)),
        ],
        out_specs=pl.BlockSpec((TC_BLK, H), lambda i: (i, 0)),
        out_shape=jax.ShapeDtypeStruct((NPAD, H), jnp.float32),
    )(h, wout, bout)


SUB = 5
SUBS = GROUPS // SUB
NTILE = NPAD // 16


def _sc_body(xl_hbm, al_hbm, ar_hbm, ends_hbm, att_hbm, out_hbm,
             ends_v, ar_v, al_v, rows_v, att_v, wbuf_v, out_v, xsh_v, sem0):
    c = lax.axis_index("c")
    s = lax.axis_index("s")
    wid = s * 2 + c

    pltpu.sync_copy(
        xl_hbm.at[pl.ds(s * NTILE, NTILE)],
        xsh_v.at[pl.ds(s * NTILE, NTILE)],
    )
    pltpu.sync_copy(ar_hbm, ar_v)
    pltpu.sync_copy(att_hbm, att_v)
    plsc.subcore_barrier()

    zeros16 = jnp.zeros((16,), jnp.int32)
    lanes = lax.iota(jnp.int32, 16)
    att0 = att_v[0, :]

    def sub_body(sub, _):
        gb = wid * GROUPS + sub * SUB
        nb = gb * GROUP
        pltpu.sync_copy(ends_hbm.at[pl.ds(gb, SUB)], ends_v)
        pltpu.sync_copy(al_hbm.at[pl.ds(nb, SUB * GROUP)], al_v)
        pltpu.sync_copy(xl_hbm.at[pl.ds(nb, SUB * GROUP)], out_v)

        def scale_body(r):
            for cc in range(H // 16):
                out_v[r, pl.ds(cc * 16, 16)] = (
                    out_v[r, pl.ds(cc * 16, 16)] * att0
                )

        plsc.parallel_loop(0, SUB * GROUP, 1, unroll=2)(scale_body)

        def step_body(t, _):
            g = t // K
            hop = t - g * K
            pltpu.async_copy(
                xsh_v.at[ends_v.at[g, hop]], rows_v, sem0
            ).wait()

            atth = att_v[hop + 1, :] * (1.0 / RWS)
            alv = al_v[pl.ds(g * GROUP, GROUP)]
            svs = []
            for w in range(RWS):
                pos = lanes * RWS + w
                idxw = plsc.load_gather(
                    ends_v, [zeros16 + g, zeros16 + hop, pos]
                )
                arw = plsc.load_gather(ar_v, [idxw])
                e = alv + arw
                svs.append(jnp.maximum(e, 0.2 * e))
            m = svs[0]
            for w in range(1, RWS):
                m = jnp.maximum(m, svs[w])
            ps = [jnp.exp(sv - m) for sv in svs]
            z = ps[0]
            for w in range(1, RWS):
                z = z + ps[w]
            coef = atth / z
            for w in range(RWS):
                wbuf_v[pl.ds(w * 16, 16)] = ps[w] * coef

            def n_body(n):
                row = g * GROUP + n
                wsp = [
                    plsc.load_gather(wbuf_v, [zeros16 + (w * 16) + n])
                    for w in range(RWS)
                ]
                for cc in range(H // 16):
                    t = [
                        wsp[w] * rows_v[n * RWS + w, pl.ds(cc * 16, 16)]
                        for w in range(RWS)
                    ]
                    t = [t[0] + t[1], t[2] + t[3], t[4] + t[5], t[6] + t[7]]
                    acc = (t[0] + t[1]) + (t[2] + t[3])
                    plsc.addupdate(out_v.at[row, pl.ds(cc * 16, 16)], acc)

            plsc.parallel_loop(0, GROUP, 1, unroll=2)(n_body)
            return 0

        lax.fori_loop(0, K * SUB, step_body, 0)
        pltpu.sync_copy(out_v, out_hbm.at[pl.ds(nb, SUB * GROUP)])
        return 0

    lax.fori_loop(0, SUBS, sub_body, 0)


@functools.lru_cache(maxsize=None)
def _sc_aggregate():
    return pl.kernel(
        _sc_body,
        out_type=jax.ShapeDtypeStruct((NPAD, H), jnp.float32),
        mesh=plsc.VectorSubcoreMesh(
            core_axis_name="c",
            subcore_axis_name="s",
            num_cores=2,
            num_subcores=16,
        ),
        scratch_types=[
            pltpu.VMEM((SUB, K, GROW), jnp.int32),
            pltpu.VMEM((NPAD,), jnp.float32),
            pltpu.VMEM((SUB * GROUP,), jnp.float32),
            pltpu.VMEM((GROW, H), jnp.float32),
            pltpu.VMEM((8, 16), jnp.float32),
            pltpu.VMEM((RWS * 16,), jnp.float32),
            pltpu.VMEM((SUB * GROUP, H), jnp.float32),
            pltpu.VMEM_SHARED((NPAD, H), jnp.float32),
            pltpu.SemaphoreType.DMA,
        ],
        compiler_params=pltpu.CompilerParams(needs_layout_passes=False),
    )


def _layer(xp, ends1d, attp, w, b, walr, balr):
    xl, alr = _tc_linear(xp, w, b, walr, balr)
    al = alr[:, 0]
    ar = alr[:, 1]
    return _sc_aggregate()(xl, al, ar, ends1d, attp)


def kernel(x, edge_index, walk_ends, att, W0, b0, Wal0, bal0, War0, bar0,
           W1, b1, Wal1, bal1, War1, bar1, Wout, bout):
    xp = jnp.zeros((NPAD, D), jnp.float32).at[:N].set(x)

    ends = walk_ends.astype(jnp.int32).reshape(2 * K, N * RWS)
    ends_pad = jnp.zeros((2 * K, EPN), jnp.int32).at[:, : N * RWS].set(ends)
    ends0 = jnp.transpose(
        ends_pad[:K].reshape(K, NPAD // GROUP, GROW), (1, 0, 2)
    )
    ends1 = jnp.transpose(
        ends_pad[K:].reshape(K, NPAD // GROUP, GROW), (1, 0, 2)
    )

    attp0 = jnp.zeros((8, 16), jnp.float32).at[: K + 1].set(
        jnp.tile(att[0][:, None], (1, 16))
    )
    attp1 = jnp.zeros((8, 16), jnp.float32).at[: K + 1].set(
        jnp.tile(att[1][:, None], (1, 16))
    )

    def packw(wal, bal, war, bar):
        walr = jnp.zeros((H, H), jnp.float32)
        walr = walr.at[:, 0].set(wal[:, 0]).at[:, 1].set(war[:, 0])
        balr = jnp.zeros((1, H), jnp.float32)
        balr = balr.at[0, 0].set(bal[0]).at[0, 1].set(bar[0])
        return walr, balr

    walr0, balr0 = packw(Wal0, bal0, War0, bar0)
    walr1, balr1 = packw(Wal1, bal1, War1, bar1)

    h = _layer(xp, ends0, attp0, W0, b0.reshape(1, H), walr0, balr0)
    h = _layer(h, ends1, attp1, W1, b1.reshape(1, H), walr1, balr1)

    woutp = jnp.zeros((H, H), jnp.float32).at[:, :C].set(Wout)
    boutp = jnp.zeros((1, H), jnp.float32).at[0, :C].set(bout)
    out = _tc_out(h, woutp, boutp)
    return out[:N, :C]


# speedup vs baseline: 1.3706x; 1.3245x over previous
import functools

import jax
import jax.numpy as jnp
from jax import lax
from jax.experimental import pallas as pl
from jax.experimental.pallas import tpu as pltpu
from jax.experimental.pallas import tpu_sc as plsc

N = 10000
D = 128
H = 128
C = 16
K = 4
RWS = 8

NW = 32
NPW = 320
NPAD = NW * NPW
GROUP = 16
GROUPS = NPW // GROUP
GROW = GROUP * RWS
EPN = NPAD * RWS

TC_BLK = 1024


def _lin_body(x_ref, w_ref, b_ref, walr_ref, balr_ref, xl_ref, alr_ref):
    t = jnp.dot(x_ref[...], w_ref[...], preferred_element_type=jnp.float32)
    t = t + b_ref[...]
    xl_ref[...] = t
    alr_ref[...] = (
        jnp.dot(t, walr_ref[...], preferred_element_type=jnp.float32)
        + balr_ref[...]
    )


def _tc_linear(xp, w, b, walr, balr):
    grid = (NPAD // TC_BLK,)
    return pl.pallas_call(
        _lin_body,
        grid=grid,
        in_specs=[
            pl.BlockSpec((TC_BLK, D), lambda i: (i, 0)),
            pl.BlockSpec((D, H), lambda i: (0, 0)),
            pl.BlockSpec((1, H), lambda i: (0, 0)),
            pl.BlockSpec((H, H), lambda i: (0, 0)),
            pl.BlockSpec((1, H), lambda i: (0, 0)),
        ],
        out_specs=[
            pl.BlockSpec((TC_BLK, H), lambda i: (i, 0)),
            pl.BlockSpec((TC_BLK, H), lambda i: (i, 0)),
        ],
        out_shape=[
            jax.ShapeDtypeStruct((NPAD, H), jnp.float32),
            jax.ShapeDtypeStruct((NPAD, H), jnp.float32),
        ],
    )(xp, w, b, walr, balr)


def _out_body(h_ref, w_ref, b_ref, o_ref):
    z = jnp.dot(h_ref[...], w_ref[...], preferred_element_type=jnp.float32)
    z = z + b_ref[...]
    col = lax.broadcasted_iota(jnp.int32, z.shape, 1)
    zm = jnp.where(col < C, z, -jnp.inf)
    mx = jnp.max(zm, axis=1, keepdims=True)
    lse = jnp.log(jnp.sum(jnp.exp(zm - mx), axis=1, keepdims=True)) + mx
    o_ref[...] = z - lse


def _tc_out(h, wout, bout):
    grid = (NPAD // TC_BLK,)
    return pl.pallas_call(
        _out_body,
        grid=grid,
        in_specs=[
            pl.BlockSpec((TC_BLK, H), lambda i: (i, 0)),
            pl.BlockSpec((H, H), lambda i: (0, 0)),
            pl.BlockSpec((1, H), lambda i: (0, 0)),
        ],
        out_specs=pl.BlockSpec((TC_BLK, H), lambda i: (i, 0)),
        out_shape=jax.ShapeDtypeStruct((NPAD, H), jnp.float32),
    )(h, wout, bout)


SUB = 5
SUBS = GROUPS // SUB
NTILE = NPAD // 16


def _sc_body(xl_hbm, al_hbm, ar_hbm, ends_hbm, att_hbm, out_hbm,
             ends_v, ar_v, al_v, rows_v, att_v, wbuf_v, out_v, xsh_v,
             sem0, sem1):
    c = lax.axis_index("c")
    s = lax.axis_index("s")
    wid = s * 2 + c

    pltpu.sync_copy(
        xl_hbm.at[pl.ds(s * NTILE, NTILE)],
        xsh_v.at[pl.ds(s * NTILE, NTILE)],
    )
    pltpu.sync_copy(ar_hbm, ar_v)
    pltpu.sync_copy(att_hbm, att_v)
    plsc.subcore_barrier()

    zeros16 = jnp.zeros((16,), jnp.int32)
    lanes = lax.iota(jnp.int32, 16)
    att0 = att_v[0, :]
    sems = (sem0, sem1)

    def sub_body(sub, _):
        gb = wid * GROUPS + sub * SUB
        nb = gb * GROUP
        pltpu.sync_copy(ends_hbm.at[pl.ds(gb, SUB)], ends_v)
        pltpu.sync_copy(al_hbm.at[pl.ds(nb, SUB * GROUP)], al_v)
        pltpu.sync_copy(xl_hbm.at[pl.ds(nb, SUB * GROUP)], out_v)

        def scale_body(r):
            for cc in range(H // 16):
                out_v[r, pl.ds(cc * 16, 16)] = (
                    out_v[r, pl.ds(cc * 16, 16)] * att0
                )

        plsc.parallel_loop(0, SUB * GROUP, 1, unroll=2)(scale_body)

        def half_step(t2, b):
            g = t2 // 8
            rem = t2 - g * 8
            hop = rem // 2
            half = rem - hop * 2

            @pl.when(t2 + 1 < K * SUB * 2)
            def _():
                g2 = (t2 + 1) // 8
                rem2 = (t2 + 1) - g2 * 8
                hop2 = rem2 // 2
                half2 = rem2 - hop2 * 2
                pltpu.async_copy(
                    xsh_v.at[ends_v.at[g2, hop2, pl.ds(half2 * 64, 64)]],
                    rows_v.at[1 - b],
                    sems[1 - b],
                )

            pltpu.make_async_copy(
                xl_hbm.at[pl.ds(0, 64)], rows_v.at[b], sems[b]
            ).wait()

            @pl.when(half == 0)
            def _():
                atth = att_v[hop + 1, :] * (1.0 / RWS)
                alv = al_v[pl.ds(g * GROUP, GROUP)]
                svs = []
                for w in range(RWS):
                    pos = lanes * RWS + w
                    idxw = plsc.load_gather(
                        ends_v, [zeros16 + g, zeros16 + hop, pos]
                    )
                    arw = plsc.load_gather(ar_v, [idxw])
                    e = alv + arw
                    svs.append(jnp.maximum(e, 0.2 * e))
                m = svs[0]
                for w in range(1, RWS):
                    m = jnp.maximum(m, svs[w])
                ps = [jnp.exp(sv - m) for sv in svs]
                z = ps[0]
                for w in range(1, RWS):
                    z = z + ps[w]
                coef = atth / z
                for w in range(RWS):
                    wbuf_v[pl.ds(w * 16, 16)] = ps[w] * coef

            def n_body(n):
                row = g * GROUP + half * 8 + n
                wsp = [
                    plsc.load_gather(
                        wbuf_v, [zeros16 + (w * 16) + (half * 8) + n]
                    )
                    for w in range(RWS)
                ]
                for cc in range(H // 16):
                    t = [
                        wsp[w] * rows_v[b, n * RWS + w, pl.ds(cc * 16, 16)]
                        for w in range(RWS)
                    ]
                    t = [t[0] + t[1], t[2] + t[3], t[4] + t[5], t[6] + t[7]]
                    acc = (t[0] + t[1]) + (t[2] + t[3])
                    plsc.addupdate(out_v.at[row, pl.ds(cc * 16, 16)], acc)

            plsc.parallel_loop(0, 8, 1, unroll=2)(n_body)

        pltpu.async_copy(
            xsh_v.at[ends_v.at[0, 0, pl.ds(0, 64)]], rows_v.at[0], sem0
        )

        def pair_body(p, _):
            for b in range(2):
                half_step(p * 2 + b, b)
            return 0

        lax.fori_loop(0, K * SUB, pair_body, 0)
        pltpu.sync_copy(out_v, out_hbm.at[pl.ds(nb, SUB * GROUP)])
        return 0

    lax.fori_loop(0, SUBS, sub_body, 0)


@functools.lru_cache(maxsize=None)
def _sc_aggregate():
    return pl.kernel(
        _sc_body,
        out_type=jax.ShapeDtypeStruct((NPAD, H), jnp.float32),
        mesh=plsc.VectorSubcoreMesh(
            core_axis_name="c",
            subcore_axis_name="s",
            num_cores=2,
            num_subcores=16,
        ),
        scratch_types=[
            pltpu.VMEM((SUB, K, GROW), jnp.int32),
            pltpu.VMEM((NPAD,), jnp.float32),
            pltpu.VMEM((SUB * GROUP,), jnp.float32),
            pltpu.VMEM((2, 64, H), jnp.float32),
            pltpu.VMEM((8, 16), jnp.float32),
            pltpu.VMEM((RWS * 16,), jnp.float32),
            pltpu.VMEM((SUB * GROUP, H), jnp.float32),
            pltpu.VMEM_SHARED((NPAD, H), jnp.float32),
            pltpu.SemaphoreType.DMA,
            pltpu.SemaphoreType.DMA,
        ],
        compiler_params=pltpu.CompilerParams(needs_layout_passes=False),
    )


def _layer(xp, ends1d, attp, w, b, walr, balr):
    xl, alr = _tc_linear(xp, w, b, walr, balr)
    al = alr[:, 0]
    ar = alr[:, 1]
    return _sc_aggregate()(xl, al, ar, ends1d, attp)


def kernel(x, edge_index, walk_ends, att, W0, b0, Wal0, bal0, War0, bar0,
           W1, b1, Wal1, bal1, War1, bar1, Wout, bout):
    xp = jnp.zeros((NPAD, D), jnp.float32).at[:N].set(x)

    ends = walk_ends.astype(jnp.int32).reshape(2 * K, N * RWS)
    ends_pad = jnp.zeros((2 * K, EPN), jnp.int32).at[:, : N * RWS].set(ends)
    ends0 = jnp.transpose(
        ends_pad[:K].reshape(K, NPAD // GROUP, GROW), (1, 0, 2)
    )
    ends1 = jnp.transpose(
        ends_pad[K:].reshape(K, NPAD // GROUP, GROW), (1, 0, 2)
    )

    attp0 = jnp.zeros((8, 16), jnp.float32).at[: K + 1].set(
        jnp.tile(att[0][:, None], (1, 16))
    )
    attp1 = jnp.zeros((8, 16), jnp.float32).at[: K + 1].set(
        jnp.tile(att[1][:, None], (1, 16))
    )

    def packw(wal, bal, war, bar):
        walr = jnp.zeros((H, H), jnp.float32)
        walr = walr.at[:, 0].set(wal[:, 0]).at[:, 1].set(war[:, 0])
        balr = jnp.zeros((1, H), jnp.float32)
        balr = balr.at[0, 0].set(bal[0]).at[0, 1].set(bar[0])
        return walr, balr

    walr0, balr0 = packw(Wal0, bal0, War0, bar0)
    walr1, balr1 = packw(Wal1, bal1, War1, bar1)

    h = _layer(xp, ends0, attp0, W0, b0.reshape(1, H), walr0, balr0)
    h = _layer(h, ends1, attp1, W1, b1.reshape(1, H), walr1, balr1)

    woutp = jnp.zeros((H, H), jnp.float32).at[:, :C].set(Wout)
    boutp = jnp.zeros((1, H), jnp.float32).at[0, :C].set(bout)
    out = _tc_out(h, woutp, boutp)
    return out[:N, :C]
